# baseline (speedup 1.0000x reference)
import jax
import jax.numpy as jnp
from jax import lax
from jax.experimental import pallas as pl
from jax.experimental.pallas import tpu as pltpu
from jax.experimental.pallas import tpu_sc as plsc

BATCH = 16384
DIM = 64
WIDE = 128
LANES = 16
NUM_CORES = 2
NUM_SUBCORES = 16
NUM_WORKERS = NUM_CORES * NUM_SUBCORES
ROWS_PER_WORKER = BATCH // NUM_WORKERS
CHUNK = 128
NCHUNK = ROWS_PER_WORKER // CHUNK
GROUP = 16


def _dict_embed_kernel(user_hbm, pu_hbm, item_hbm, pi_hbm, ctx_hbm, pc_hbm,
                       wu_hbm, wi_hbm, wc_hbm,
                       out_hbm, idx_u, idx_i, idx_c, par_u, par_i, par_c,
                       rows_u, rows_i, rows_c,
                       sem_u, sem_i, sem_c, sem_out):
    wid = lax.axis_index("s") * NUM_CORES + lax.axis_index("c")
    chunk_base = wid * NCHUNK
    row_base = wid * ROWS_PER_WORKER

    for (src, dst) in ((user_hbm, idx_u), (item_hbm, idx_i), (ctx_hbm, idx_c),
                       (pu_hbm, par_u), (pi_hbm, par_i), (pc_hbm, par_c)):
        pltpu.sync_copy(src.at[pl.ds(chunk_base, NCHUNK)], dst)

    def fire(j):
        sl = pl.ds((j % 2) * CHUNK, CHUNK)
        return (pltpu.async_copy(wu_hbm.at[idx_u.at[j]], rows_u.at[sl], sem_u),
                pltpu.async_copy(wi_hbm.at[idx_i.at[j]], rows_i.at[sl], sem_i),
                pltpu.async_copy(wc_hbm.at[idx_c.at[j]], rows_c.at[sl], sem_c))

    gathers = {0: fire(0)}
    stores = {}
    for j in range(NCHUNK):
        if j + 1 < NCHUNK:
            if j - 1 in stores:
                stores.pop(j - 1).wait()
            gathers[j + 1] = fire(j + 1)
        for c in gathers.pop(j):
            c.wait()
        base = (j % 2) * CHUNK

        def body(g, _):
            r0 = base + g * GROUP
            pv_u = par_u[j, pl.ds(g * GROUP, GROUP)]
            pv_i = par_i[j, pl.ds(g * GROUP, GROUP)]
            pv_c = par_c[j, pl.ds(g * GROUP, GROUP)]
            for k in range(GROUP):
                du = pv_u[k] * DIM
                di = pv_i[k] * DIM
                dc = pv_c[k] * DIM
                for c in range(DIM // LANES):
                    o = c * LANES
                    s = (rows_u[r0 + k, pl.ds(du + o, LANES)]
                         + rows_i[r0 + k, pl.ds(di + o, LANES)]
                         + rows_c[r0 + k, pl.ds(dc + o, LANES)])
                    rows_u[r0 + k, pl.ds(o, LANES)] = s
            return _

        lax.fori_loop(0, CHUNK // GROUP, body, 0)

        stores[j] = pltpu.async_copy(
            rows_u.at[pl.ds(base, CHUNK)],
            out_hbm.at[pl.ds(row_base + j * CHUNK, CHUNK)],
            sem_out)
    for s in stores.values():
        s.wait()


def _dict_embed(u2, pu2, i2, pi2, c2, pc2, wu, wi, wc):
    mesh = plsc.VectorSubcoreMesh(core_axis_name="c", subcore_axis_name="s")
    return pl.kernel(
        _dict_embed_kernel,
        mesh=mesh,
        out_type=jax.ShapeDtypeStruct((BATCH, WIDE), jnp.float32),
        scratch_types=[
            pltpu.VMEM((NCHUNK, CHUNK), jnp.int32),
            pltpu.VMEM((NCHUNK, CHUNK), jnp.int32),
            pltpu.VMEM((NCHUNK, CHUNK), jnp.int32),
            pltpu.VMEM((NCHUNK, CHUNK), jnp.int32),
            pltpu.VMEM((NCHUNK, CHUNK), jnp.int32),
            pltpu.VMEM((NCHUNK, CHUNK), jnp.int32),
            pltpu.VMEM((2 * CHUNK, WIDE), jnp.float32),
            pltpu.VMEM((2 * CHUNK, WIDE), jnp.float32),
            pltpu.VMEM((2 * CHUNK, WIDE), jnp.float32),
            pltpu.SemaphoreType.DMA,
            pltpu.SemaphoreType.DMA,
            pltpu.SemaphoreType.DMA,
            pltpu.SemaphoreType.DMA,
        ],
    )(u2, pu2, i2, pi2, c2, pc2, wu, wi, wc)


def _pack_body(lo_ref, hi_ref, out_ref):
    out_ref[:, 0:DIM] = lo_ref[...].T
    out_ref[:, DIM:WIDE] = hi_ref[...].T


def _pack(w, block):
    v = w.shape[0]
    half = v // 2
    nblk = half // block
    wt = w.T
    return pl.pallas_call(
        _pack_body,
        grid=(nblk,),
        in_specs=[
            pl.BlockSpec((DIM, block), lambda j: (0, j)),
            pl.BlockSpec((DIM, block), lambda j, n=nblk: (0, j + n)),
        ],
        out_specs=pl.BlockSpec((block, WIDE), lambda j: (j, 0)),
        out_shape=jax.ShapeDtypeStruct((half, WIDE), jnp.float32),
    )(wt, wt)


def _pack_body(x_ref, out_ref):
    y = x_ref[...].T
    h = y.shape[0] // 2
    out_ref[:, 0:DIM] = y[0:h]
    out_ref[:, DIM:WIDE] = y[h:2 * h]


def _pack_main(wt, nblk, block=1024):
    half = wt.shape[1] // 2
    return pl.pallas_call(
        _pack_body,
        grid=(nblk,),
        in_specs=[pl.BlockSpec((DIM, block), lambda j: (0, j))],
        out_specs=pl.BlockSpec((block // 2, WIDE), lambda j: (j, 0)),
        out_shape=jax.ShapeDtypeStruct((half, WIDE), jnp.float32),
    )(wt)


def _pack_whole(wt):
    return pl.pallas_call(
        _pack_body,
        out_shape=jax.ShapeDtypeStruct((wt.shape[1] // 2, WIDE), jnp.float32),
    )(wt)


_B = 1024


def _prep_table(w):
    v = w.shape[0]
    nblk = v // _B
    cov = nblk * _B
    wt = w.T
    if v <= _B:
        packed, cov = _pack_whole(wt), v
    elif cov == v:
        packed = _pack_main(wt, nblk)
    else:
        main = _pack_main(wt, nblk)
        tail = _pack_whole(w[cov:].T)
        packed = jax.lax.dynamic_update_slice(main, tail, (cov // 2, 0))
    return packed, cov


def _map_idx(idx, vocab, cov):
    v = idx.astype(jnp.int32)
    line_m = (v >> 10) * (_B // 2) + (v & (_B // 2 - 1))
    par_m = (v >> 9) & 1
    if cov == vocab:
        if vocab <= _B:
            half = vocab // 2
            line_m = v % half
            par_m = v // half
        line, par = line_m, par_m
    else:
        t = v - cov
        th = (vocab - cov) // 2
        line_t = cov // 2 + jnp.where(t >= th, t - th, t)
        par_t = (t >= th).astype(jnp.int32)
        tail = v >= cov
        line = jnp.where(tail, line_t, line_m)
        par = jnp.where(tail, par_t, par_m)
    return (line.reshape(BATCH // CHUNK, CHUNK),
            par.reshape(BATCH // CHUNK, CHUNK))


def kernel(user, item, context, W_user, W_item, W_context):
    wu, cov_u = _prep_table(W_user)
    wi, cov_i = _prep_table(W_item)
    wc, cov_c = _prep_table(W_context)
    u2, pu2 = _map_idx(user, 1000000, cov_u)
    i2, pi2 = _map_idx(item, 100000, cov_i)
    c2, pc2 = _map_idx(context, 1000, cov_c)
    out = _dict_embed(u2, pu2, i2, pi2, c2, pc2, wu, wi, wc)
    return out[:, :DIM]


# speedup vs baseline: 1.4468x; 1.4468x over previous
import jax
import jax.numpy as jnp
from jax import lax
from jax.experimental import pallas as pl
from jax.experimental.pallas import tpu as pltpu
from jax.experimental.pallas import tpu_sc as plsc

BATCH = 16384
DIM = 64
WIDE = 128
LANES = 16
NUM_CORES = 2
NUM_SUBCORES = 16
NUM_WORKERS = NUM_CORES * NUM_SUBCORES
ROWS_PER_WORKER = BATCH // NUM_WORKERS
CHUNK = 128
NCHUNK = ROWS_PER_WORKER // CHUNK
GROUP = 16


def _dict_embed_kernel(user_hbm, pu_hbm, item_hbm, pi_hbm, ctx_hbm, pc_hbm,
                       wu_hbm, wi_hbm, wc_hbm,
                       out_hbm, idx_u, idx_i, idx_c, par_u, par_i, par_c,
                       rows_u, rows_i, rows_c,
                       sem_u, sem_i, sem_c, sem_out):
    wid = lax.axis_index("s") * NUM_CORES + lax.axis_index("c")
    chunk_base = wid * NCHUNK
    row_base = wid * ROWS_PER_WORKER

    for (src, dst) in ((user_hbm, idx_u), (item_hbm, idx_i), (ctx_hbm, idx_c),
                       (pu_hbm, par_u), (pi_hbm, par_i), (pc_hbm, par_c)):
        pltpu.sync_copy(src.at[pl.ds(chunk_base, NCHUNK)], dst)

    def fire(j):
        sl = pl.ds((j % 2) * CHUNK, CHUNK)
        return (pltpu.async_copy(wu_hbm.at[idx_u.at[j]], rows_u.at[sl], sem_u),
                pltpu.async_copy(wi_hbm.at[idx_i.at[j]], rows_i.at[sl], sem_i),
                pltpu.async_copy(wc_hbm.at[idx_c.at[j]], rows_c.at[sl], sem_c))

    gathers = {0: fire(0)}
    stores = {}
    for j in range(NCHUNK):
        if j + 1 < NCHUNK:
            if j - 1 in stores:
                stores.pop(j - 1).wait()
            gathers[j + 1] = fire(j + 1)
        for c in gathers.pop(j):
            c.wait()
        base = (j % 2) * CHUNK

        def body(g, _):
            r0 = base + g * GROUP
            pv_u = par_u[j, pl.ds(g * GROUP, GROUP)]
            pv_i = par_i[j, pl.ds(g * GROUP, GROUP)]
            pv_c = par_c[j, pl.ds(g * GROUP, GROUP)]
            for k in range(GROUP):
                du = pv_u[k] * DIM
                di = pv_i[k] * DIM
                dc = pv_c[k] * DIM
                for c in range(DIM // LANES):
                    o = c * LANES
                    s = (rows_u[r0 + k, pl.ds(du + o, LANES)]
                         + rows_i[r0 + k, pl.ds(di + o, LANES)]
                         + rows_c[r0 + k, pl.ds(dc + o, LANES)])
                    rows_u[r0 + k, pl.ds(o, LANES)] = s
            return _

        lax.fori_loop(0, CHUNK // GROUP, body, 0)

        stores[j] = pltpu.async_copy(
            rows_u.at[pl.ds(base, CHUNK)],
            out_hbm.at[pl.ds(row_base + j * CHUNK, CHUNK)],
            sem_out)
    for s in stores.values():
        s.wait()


def _dict_embed(u2, pu2, i2, pi2, c2, pc2, wu, wi, wc):
    mesh = plsc.VectorSubcoreMesh(core_axis_name="c", subcore_axis_name="s")
    return pl.kernel(
        _dict_embed_kernel,
        mesh=mesh,
        out_type=jax.ShapeDtypeStruct((BATCH, WIDE), jnp.float32),
        scratch_types=[
            pltpu.VMEM((NCHUNK, CHUNK), jnp.int32),
            pltpu.VMEM((NCHUNK, CHUNK), jnp.int32),
            pltpu.VMEM((NCHUNK, CHUNK), jnp.int32),
            pltpu.VMEM((NCHUNK, CHUNK), jnp.int32),
            pltpu.VMEM((NCHUNK, CHUNK), jnp.int32),
            pltpu.VMEM((NCHUNK, CHUNK), jnp.int32),
            pltpu.VMEM((2 * CHUNK, WIDE), jnp.float32),
            pltpu.VMEM((2 * CHUNK, WIDE), jnp.float32),
            pltpu.VMEM((2 * CHUNK, WIDE), jnp.float32),
            pltpu.SemaphoreType.DMA,
            pltpu.SemaphoreType.DMA,
            pltpu.SemaphoreType.DMA,
            pltpu.SemaphoreType.DMA,
        ],
    )(u2, pu2, i2, pi2, c2, pc2, wu, wi, wc)


def _pack_body(x_ref, out_ref):
    x = x_ref[...]
    d = x.shape[0]
    eye = jnp.eye(d, dtype=jnp.float32)
    y = jax.lax.dot_general(x, eye, (((0,), (0,)), ((), ())),
                            preferred_element_type=jnp.float32)
    h = y.shape[0] // 2
    out_ref[:, 0:DIM] = y[0:h]
    out_ref[:, DIM:WIDE] = y[h:2 * h]


def _pack_main(wt, nblk, block):
    half = wt.shape[1] // 2
    return pl.pallas_call(
        _pack_body,
        grid=(nblk,),
        in_specs=[pl.BlockSpec((DIM, block), lambda j: (0, j))],
        out_specs=pl.BlockSpec((block // 2, WIDE), lambda j: (j, 0)),
        out_shape=jax.ShapeDtypeStruct((half, WIDE), jnp.float32),
    )(wt)


def _pack_whole(wt):
    return pl.pallas_call(
        _pack_body,
        out_shape=jax.ShapeDtypeStruct((wt.shape[1] // 2, WIDE), jnp.float32),
    )(wt)


_B = 2048


def _prep_table(w):
    v = w.shape[0]
    nblk = v // _B
    cov = nblk * _B
    wt = w.T
    if v <= _B:
        packed, cov = _pack_whole(wt), v
    elif cov == v:
        packed = _pack_main(wt, nblk, _B)
    else:
        main = _pack_main(wt, nblk, _B)
        tail = _pack_whole(w[cov:].T)
        packed = jax.lax.dynamic_update_slice(main, tail, (cov // 2, 0))
    return packed, cov


def _map_idx(idx, vocab, cov):
    v = idx.astype(jnp.int32)
    hb = _B // 2
    if cov == vocab and vocab <= _B:
        half = vocab // 2
        line = v % half
        par = v // half
    else:
        line_m = (v // _B) * hb + (v % hb)
        par_m = (v % _B) // hb
        if cov == vocab:
            line, par = line_m, par_m
        else:
            t = v - cov
            th = (vocab - cov) // 2
            line_t = cov // 2 + jnp.where(t >= th, t - th, t)
            par_t = (t >= th).astype(jnp.int32)
            tail = v >= cov
            line = jnp.where(tail, line_t, line_m)
            par = jnp.where(tail, par_t, par_m)
    return (line.reshape(BATCH // CHUNK, CHUNK),
            par.reshape(BATCH // CHUNK, CHUNK))


def kernel(user, item, context, W_user, W_item, W_context):
    wu, cov_u = _prep_table(W_user)
    wi, cov_i = _prep_table(W_item)
    wc, cov_c = _prep_table(W_context)
    u2, pu2 = _map_idx(user, 1000000, cov_u)
    i2, pi2 = _map_idx(item, 100000, cov_i)
    c2, pc2 = _map_idx(context, 1000, cov_c)
    out = _dict_embed(u2, pu2, i2, pi2, c2, pc2, wu, wi, wc)
    return out[:, :DIM]


# speedup vs baseline: 1.5572x; 1.0763x over previous
import jax
import jax.numpy as jnp
from jax import lax
from jax.experimental import pallas as pl
from jax.experimental.pallas import tpu as pltpu
from jax.experimental.pallas import tpu_sc as plsc

BATCH = 16384
DIM = 64
LINE = 256
SUB = LINE // DIM
LANES = 16
NUM_CORES = 2
NUM_SUBCORES = 16
NUM_WORKERS = NUM_CORES * NUM_SUBCORES
ROWS_PER_WORKER = BATCH // NUM_WORKERS
CHUNK = 64
NCHUNK = ROWS_PER_WORKER // CHUNK
GROUP = 16
_B = 2048


def _dict_embed_kernel(user_hbm, pu_hbm, item_hbm, pi_hbm, ctx_hbm, pc_hbm,
                       wu_hbm, wi_hbm, wc_hbm,
                       out_hbm, idx_u, idx_i, idx_c, par_u, par_i, par_c,
                       rows_u, rows_i, rows_c,
                       sem_u, sem_i, sem_c, sem_out):
    wid = lax.axis_index("s") * NUM_CORES + lax.axis_index("c")
    chunk_base = wid * NCHUNK
    row_base = wid * ROWS_PER_WORKER

    for (src, dst) in ((user_hbm, idx_u), (item_hbm, idx_i), (ctx_hbm, idx_c),
                       (pu_hbm, par_u), (pi_hbm, par_i), (pc_hbm, par_c)):
        pltpu.sync_copy(src.at[pl.ds(chunk_base, NCHUNK)], dst)

    def fire(j):
        sl = pl.ds((j % 2) * CHUNK, CHUNK)
        return (pltpu.async_copy(wu_hbm.at[idx_u.at[j]], rows_u.at[sl], sem_u),
                pltpu.async_copy(wi_hbm.at[idx_i.at[j]], rows_i.at[sl], sem_i),
                pltpu.async_copy(wc_hbm.at[idx_c.at[j]], rows_c.at[sl], sem_c))

    gathers = {0: fire(0)}
    stores = {}
    for j in range(NCHUNK):
        if j + 1 < NCHUNK:
            if j - 1 in stores:
                stores.pop(j - 1).wait()
            gathers[j + 1] = fire(j + 1)
        for c in gathers.pop(j):
            c.wait()
        base = (j % 2) * CHUNK

        def body(g, _):
            r0 = base + g * GROUP
            pv_u = par_u[j, pl.ds(g * GROUP, GROUP)]
            pv_i = par_i[j, pl.ds(g * GROUP, GROUP)]
            pv_c = par_c[j, pl.ds(g * GROUP, GROUP)]
            for k in range(GROUP):
                du = pv_u[k] * DIM
                di = pv_i[k] * DIM
                dc = pv_c[k] * DIM
                for c in range(DIM // LANES):
                    o = c * LANES
                    s = (rows_u[r0 + k, pl.ds(du + o, LANES)]
                         + rows_i[r0 + k, pl.ds(di + o, LANES)]
                         + rows_c[r0 + k, pl.ds(dc + o, LANES)])
                    rows_u[r0 + k, pl.ds(o, LANES)] = s
            return _

        lax.fori_loop(0, CHUNK // GROUP, body, 0)

        stores[j] = pltpu.async_copy(
            rows_u.at[pl.ds(base, CHUNK)],
            out_hbm.at[pl.ds(row_base + j * CHUNK, CHUNK)],
            sem_out)
    for s in stores.values():
        s.wait()


def _dict_embed(u2, pu2, i2, pi2, c2, pc2, wu, wi, wc):
    mesh = plsc.VectorSubcoreMesh(core_axis_name="c", subcore_axis_name="s")
    return pl.kernel(
        _dict_embed_kernel,
        mesh=mesh,
        out_type=jax.ShapeDtypeStruct((BATCH, LINE), jnp.float32),
        scratch_types=[
            pltpu.VMEM((NCHUNK, CHUNK), jnp.int32),
            pltpu.VMEM((NCHUNK, CHUNK), jnp.int32),
            pltpu.VMEM((NCHUNK, CHUNK), jnp.int32),
            pltpu.VMEM((NCHUNK, CHUNK), jnp.int32),
            pltpu.VMEM((NCHUNK, CHUNK), jnp.int32),
            pltpu.VMEM((NCHUNK, CHUNK), jnp.int32),
            pltpu.VMEM((2 * CHUNK, LINE), jnp.float32),
            pltpu.VMEM((2 * CHUNK, LINE), jnp.float32),
            pltpu.VMEM((2 * CHUNK, LINE), jnp.float32),
            pltpu.SemaphoreType.DMA,
            pltpu.SemaphoreType.DMA,
            pltpu.SemaphoreType.DMA,
            pltpu.SemaphoreType.DMA,
        ],
    )(u2, pu2, i2, pi2, c2, pc2, wu, wi, wc)


def _pack_body(x_ref, out_ref):
    x = x_ref[...]
    q = x.shape[1] // SUB
    x4 = jnp.concatenate([x[:, i * q:(i + 1) * q] for i in range(SUB)],
                         axis=0)
    eye = jnp.eye(LINE, dtype=jnp.float32)
    out_ref[...] = lax.dot_general(
        x4, eye, (((0,), (0,)), ((), ())),
        preferred_element_type=jnp.float32)


def _pack_main(wt, nblk, block):
    return pl.pallas_call(
        _pack_body,
        grid=(nblk,),
        in_specs=[pl.BlockSpec((DIM, block), lambda j: (0, j))],
        out_specs=pl.BlockSpec((block // SUB, LINE), lambda j: (j, 0)),
        out_shape=jax.ShapeDtypeStruct((wt.shape[1] // SUB, LINE),
                                       jnp.float32),
    )(wt)


def _pack_whole(wt):
    return pl.pallas_call(
        _pack_body,
        out_shape=jax.ShapeDtypeStruct((wt.shape[1] // SUB, LINE),
                                       jnp.float32),
    )(wt)


def _prep_table(w):
    v = w.shape[0]
    nblk = v // _B
    cov = nblk * _B
    wt = w.T
    if v <= _B:
        packed, cov = _pack_whole(wt), v
    elif cov == v:
        packed = _pack_main(wt, nblk, _B)
    else:
        main = _pack_main(wt, nblk, _B)
        tail = _pack_whole(w[cov:].T)
        packed = jax.lax.dynamic_update_slice(main, tail, (cov // SUB, 0))
    return packed, cov


def _map_idx(idx, vocab, cov):
    v = idx.astype(jnp.int32)
    hb = _B // SUB
    if cov == vocab and vocab <= _B:
        qsz = vocab // SUB
        line = v % qsz
        par = v // qsz
    else:
        line_m = (v // _B) * hb + (v % hb)
        par_m = (v % _B) // hb
        if cov == vocab:
            line, par = line_m, par_m
        else:
            t = v - cov
            th = (vocab - cov) // SUB
            line_t = cov // SUB + (t % th)
            par_t = t // th
            tail = v >= cov
            line = jnp.where(tail, line_t, line_m)
            par = jnp.where(tail, par_t, par_m)
    return (line.reshape(BATCH // CHUNK, CHUNK),
            par.reshape(BATCH // CHUNK, CHUNK))


def kernel(user, item, context, W_user, W_item, W_context):
    wu, cov_u = _prep_table(W_user)
    wi, cov_i = _prep_table(W_item)
    wc, cov_c = _prep_table(W_context)
    u2, pu2 = _map_idx(user, 1000000, cov_u)
    i2, pi2 = _map_idx(item, 100000, cov_i)
    c2, pc2 = _map_idx(context, 1000, cov_c)
    out = _dict_embed(u2, pu2, i2, pi2, c2, pc2, wu, wi, wc)
    return out[:, :DIM]


# speedup vs baseline: 2.0359x; 1.3074x over previous
import jax
import jax.numpy as jnp
from jax import lax
from jax.experimental import pallas as pl
from jax.experimental.pallas import tpu as pltpu
from jax.experimental.pallas import tpu_sc as plsc

BATCH = 16384
DIM = 64
LINE = 256
SUB = LINE // DIM
LANES = 16
NUM_CORES = 2
NUM_SUBCORES = 16
NUM_WORKERS = NUM_CORES * NUM_SUBCORES
ROWS_PER_WORKER = BATCH // NUM_WORKERS
CHUNK = 64
NCHUNK = ROWS_PER_WORKER // CHUNK
GROUP = 16
_B = 4096


def _dict_embed_kernel(user_hbm, pu_hbm, item_hbm, pi_hbm, ctx_hbm, pc_hbm,
                       wu_hbm, wi_hbm, wc_hbm,
                       out_hbm, idx_u, idx_i, idx_c, par_u, par_i, par_c,
                       rows_u, rows_i, rows_c,
                       sem_u, sem_i, sem_c, sem_out):
    wid = lax.axis_index("s") * NUM_CORES + lax.axis_index("c")
    chunk_base = wid * NCHUNK
    row_base = wid * ROWS_PER_WORKER

    for (src, dst) in ((user_hbm, idx_u), (item_hbm, idx_i), (ctx_hbm, idx_c),
                       (pu_hbm, par_u), (pi_hbm, par_i), (pc_hbm, par_c)):
        pltpu.sync_copy(src.at[pl.ds(chunk_base, NCHUNK)], dst)

    def fire(j):
        sl = pl.ds((j % 2) * CHUNK, CHUNK)
        return (pltpu.async_copy(wu_hbm.at[idx_u.at[j]], rows_u.at[sl], sem_u),
                pltpu.async_copy(wi_hbm.at[idx_i.at[j]], rows_i.at[sl], sem_i),
                pltpu.async_copy(wc_hbm.at[idx_c.at[j]], rows_c.at[sl], sem_c))

    gathers = {0: fire(0)}
    stores = {}
    for j in range(NCHUNK):
        if j + 1 < NCHUNK:
            if j - 1 in stores:
                stores.pop(j - 1).wait()
            gathers[j + 1] = fire(j + 1)
        for c in gathers.pop(j):
            c.wait()
        base = (j % 2) * CHUNK

        def body(g, _):
            r0 = base + g * GROUP
            pv_u = par_u[j, pl.ds(g * GROUP, GROUP)]
            pv_i = par_i[j, pl.ds(g * GROUP, GROUP)]
            pv_c = par_c[j, pl.ds(g * GROUP, GROUP)]
            for k in range(GROUP):
                du = pv_u[k] * DIM
                di = pv_i[k] * DIM
                dc = pv_c[k] * DIM
                for c in range(DIM // LANES):
                    o = c * LANES
                    s = (rows_u[r0 + k, pl.ds(du + o, LANES)]
                         + rows_i[r0 + k, pl.ds(di + o, LANES)]
                         + rows_c[r0 + k, pl.ds(dc + o, LANES)])
                    rows_u[r0 + k, pl.ds(o, LANES)] = s
            return _

        lax.fori_loop(0, CHUNK // GROUP, body, 0)

        stores[j] = pltpu.async_copy(
            rows_u.at[pl.ds(base, CHUNK)],
            out_hbm.at[pl.ds(row_base + j * CHUNK, CHUNK)],
            sem_out)
    for s in stores.values():
        s.wait()


def _dict_embed(u2, pu2, i2, pi2, c2, pc2, wu, wi, wc):
    mesh = plsc.VectorSubcoreMesh(core_axis_name="c", subcore_axis_name="s")
    return pl.kernel(
        _dict_embed_kernel,
        mesh=mesh,
        out_type=jax.ShapeDtypeStruct((BATCH, LINE), jnp.float32),
        scratch_types=[
            pltpu.VMEM((NCHUNK, CHUNK), jnp.int32),
            pltpu.VMEM((NCHUNK, CHUNK), jnp.int32),
            pltpu.VMEM((NCHUNK, CHUNK), jnp.int32),
            pltpu.VMEM((NCHUNK, CHUNK), jnp.int32),
            pltpu.VMEM((NCHUNK, CHUNK), jnp.int32),
            pltpu.VMEM((NCHUNK, CHUNK), jnp.int32),
            pltpu.VMEM((2 * CHUNK, LINE), jnp.float32),
            pltpu.VMEM((2 * CHUNK, LINE), jnp.float32),
            pltpu.VMEM((2 * CHUNK, LINE), jnp.float32),
            pltpu.SemaphoreType.DMA,
            pltpu.SemaphoreType.DMA,
            pltpu.SemaphoreType.DMA,
            pltpu.SemaphoreType.DMA,
        ],
    )(u2, pu2, i2, pi2, c2, pc2, wu, wi, wc)


def _pack_body(x_ref, out_ref):
    x = x_ref[...]
    q = x.shape[1] // SUB
    x4 = jnp.concatenate([x[:, i * q:(i + 1) * q] for i in range(SUB)],
                         axis=0)
    hi = x4.astype(jnp.bfloat16)
    lo = (x4 - hi.astype(jnp.float32)).astype(jnp.bfloat16)
    eye = jnp.eye(LINE, dtype=jnp.bfloat16)
    dims = (((0,), (0,)), ((), ()))
    out_ref[...] = (
        lax.dot_general(hi, eye, dims, preferred_element_type=jnp.float32)
        + lax.dot_general(lo, eye, dims, preferred_element_type=jnp.float32))


def _pack_main(wt, nblk, block):
    return pl.pallas_call(
        _pack_body,
        grid=(nblk,),
        in_specs=[pl.BlockSpec((DIM, block), lambda j: (0, j))],
        out_specs=pl.BlockSpec((block // SUB, LINE), lambda j: (j, 0)),
        out_shape=jax.ShapeDtypeStruct((wt.shape[1] // SUB, LINE),
                                       jnp.float32),
    )(wt)


def _pack_whole(wt):
    return pl.pallas_call(
        _pack_body,
        out_shape=jax.ShapeDtypeStruct((wt.shape[1] // SUB, LINE),
                                       jnp.float32),
    )(wt)


def _prep_table(w):
    v = w.shape[0]
    nblk = v // _B
    cov = nblk * _B
    wt = w.T
    if v <= _B:
        packed, cov = _pack_whole(wt), v
    elif cov == v:
        packed = _pack_main(wt, nblk, _B)
    else:
        main = _pack_main(wt, nblk, _B)
        tail = _pack_whole(w[cov:].T)
        packed = jax.lax.dynamic_update_slice(main, tail, (cov // SUB, 0))
    return packed, cov


def _map_idx(idx, vocab, cov):
    v = idx.astype(jnp.int32)
    hb = _B // SUB
    if cov == vocab and vocab <= _B:
        qsz = vocab // SUB
        line = v % qsz
        par = v // qsz
    else:
        line_m = (v // _B) * hb + (v % hb)
        par_m = (v % _B) // hb
        if cov == vocab:
            line, par = line_m, par_m
        else:
            t = v - cov
            th = (vocab - cov) // SUB
            line_t = cov // SUB + (t % th)
            par_t = t // th
            tail = v >= cov
            line = jnp.where(tail, line_t, line_m)
            par = jnp.where(tail, par_t, par_m)
    return (line.reshape(BATCH // CHUNK, CHUNK),
            par.reshape(BATCH // CHUNK, CHUNK))


def kernel(user, item, context, W_user, W_item, W_context):
    wu, cov_u = _prep_table(W_user)
    wi, cov_i = _prep_table(W_item)
    wc, cov_c = _prep_table(W_context)
    u2, pu2 = _map_idx(user, 1000000, cov_u)
    i2, pi2 = _map_idx(item, 100000, cov_i)
    c2, pc2 = _map_idx(context, 1000, cov_c)
    out = _dict_embed(u2, pu2, i2, pi2, c2, pc2, wu, wi, wc)
    return out[:, :DIM]


# speedup vs baseline: 2.1323x; 1.0473x over previous
import jax
import jax.numpy as jnp
from jax import lax
from jax.experimental import pallas as pl
from jax.experimental.pallas import tpu as pltpu
from jax.experimental.pallas import tpu_sc as plsc

BATCH = 16384
DIM = 64
LINE = 256
SUB = LINE // DIM
LANES = 16
NUM_CORES = 2
NUM_SUBCORES = 16
NUM_WORKERS = NUM_CORES * NUM_SUBCORES
ROWS_PER_WORKER = BATCH // NUM_WORKERS
CHUNK = 64
NCHUNK = ROWS_PER_WORKER // CHUNK
GROUP = 16
_B = 4096


def _dict_embed_kernel(user_hbm, pu_hbm, item_hbm, pi_hbm, ctx_hbm, pc_hbm,
                       wu_hbm, wi_hbm, wc_hbm,
                       out_hbm, idx_u, idx_i, idx_c, par_u, par_i, par_c,
                       rows_u, rows_i, rows_c,
                       sem_u, sem_i, sem_c, sem_out):
    wid = lax.axis_index("s") * NUM_CORES + lax.axis_index("c")
    chunk_base = wid * NCHUNK
    row_base = wid * ROWS_PER_WORKER

    for (src, dst) in ((user_hbm, idx_u), (item_hbm, idx_i), (ctx_hbm, idx_c),
                       (pu_hbm, par_u), (pi_hbm, par_i), (pc_hbm, par_c)):
        pltpu.sync_copy(src.at[pl.ds(chunk_base, NCHUNK)], dst)

    def fire(j):
        sl = pl.ds((j % 2) * CHUNK, CHUNK)
        return (pltpu.async_copy(wu_hbm.at[idx_u.at[j]], rows_u.at[sl], sem_u),
                pltpu.async_copy(wi_hbm.at[idx_i.at[j]], rows_i.at[sl], sem_i),
                pltpu.async_copy(wc_hbm.at[idx_c.at[j]], rows_c.at[sl], sem_c))

    gathers = {0: fire(0)}
    stores = {}
    for j in range(NCHUNK):
        if j + 1 < NCHUNK:
            if j - 1 in stores:
                stores.pop(j - 1).wait()
            gathers[j + 1] = fire(j + 1)
        for c in gathers.pop(j):
            c.wait()
        base = (j % 2) * CHUNK

        def body(g, _):
            r0 = base + g * GROUP
            pv_u = par_u[j, pl.ds(g * GROUP, GROUP)]
            pv_i = par_i[j, pl.ds(g * GROUP, GROUP)]
            pv_c = par_c[j, pl.ds(g * GROUP, GROUP)]
            for k in range(GROUP):
                du = pv_u[k] * DIM
                di = pv_i[k] * DIM
                dc = pv_c[k] * DIM
                for c in range(DIM // LANES):
                    o = c * LANES
                    s = (rows_u[r0 + k, pl.ds(du + o, LANES)]
                         + rows_i[r0 + k, pl.ds(di + o, LANES)]
                         + rows_c[r0 + k, pl.ds(dc + o, LANES)])
                    rows_u[r0 + k, pl.ds(o, LANES)] = s
            return _

        lax.fori_loop(0, CHUNK // GROUP, body, 0)

        stores[j] = pltpu.async_copy(
            rows_u.at[pl.ds(base, CHUNK)],
            out_hbm.at[pl.ds(row_base + j * CHUNK, CHUNK)],
            sem_out)
    for s in stores.values():
        s.wait()


def _dict_embed(u2, pu2, i2, pi2, c2, pc2, wu, wi, wc):
    mesh = plsc.VectorSubcoreMesh(core_axis_name="c", subcore_axis_name="s")
    return pl.kernel(
        _dict_embed_kernel,
        mesh=mesh,
        out_type=jax.ShapeDtypeStruct((BATCH, LINE), jnp.float32),
        scratch_types=[
            pltpu.VMEM((NCHUNK, CHUNK), jnp.int32),
            pltpu.VMEM((NCHUNK, CHUNK), jnp.int32),
            pltpu.VMEM((NCHUNK, CHUNK), jnp.int32),
            pltpu.VMEM((NCHUNK, CHUNK), jnp.int32),
            pltpu.VMEM((NCHUNK, CHUNK), jnp.int32),
            pltpu.VMEM((NCHUNK, CHUNK), jnp.int32),
            pltpu.VMEM((2 * CHUNK, LINE), jnp.float32),
            pltpu.VMEM((2 * CHUNK, LINE), jnp.float32),
            pltpu.VMEM((2 * CHUNK, LINE), jnp.float32),
            pltpu.SemaphoreType.DMA,
            pltpu.SemaphoreType.DMA,
            pltpu.SemaphoreType.DMA,
            pltpu.SemaphoreType.DMA,
        ],
    )(u2, pu2, i2, pi2, c2, pc2, wu, wi, wc)


def _pack_body(x_ref, out_ref):
    x = x_ref[...]
    q = x.shape[1] // SUB
    x4 = jnp.concatenate([x[:, i * q:(i + 1) * q] for i in range(SUB)],
                         axis=0)
    hi = x4.astype(jnp.bfloat16)
    eye = jnp.eye(LINE, dtype=jnp.bfloat16)
    dims = (((0,), (0,)), ((), ()))
    out_ref[...] = lax.dot_general(hi, eye, dims,
                                   preferred_element_type=jnp.float32)


def _pack_main(wt, nblk, block):
    return pl.pallas_call(
        _pack_body,
        grid=(nblk,),
        in_specs=[pl.BlockSpec((DIM, block), lambda j: (0, j))],
        out_specs=pl.BlockSpec((block // SUB, LINE), lambda j: (j, 0)),
        out_shape=jax.ShapeDtypeStruct((wt.shape[1] // SUB, LINE),
                                       jnp.float32),
    )(wt)


def _pack_whole(wt):
    return pl.pallas_call(
        _pack_body,
        out_shape=jax.ShapeDtypeStruct((wt.shape[1] // SUB, LINE),
                                       jnp.float32),
    )(wt)


def _prep_table(w):
    v = w.shape[0]
    nblk = v // _B
    cov = nblk * _B
    wt = w.T
    if v <= _B:
        packed, cov = _pack_whole(wt), v
    elif cov == v:
        packed = _pack_main(wt, nblk, _B)
    else:
        main = _pack_main(wt, nblk, _B)
        tail = _pack_whole(w[cov:].T)
        packed = jax.lax.dynamic_update_slice(main, tail, (cov // SUB, 0))
    return packed, cov


def _map_idx(idx, vocab, cov):
    v = idx.astype(jnp.int32)
    hb = _B // SUB
    if cov == vocab and vocab <= _B:
        qsz = vocab // SUB
        line = v % qsz
        par = v // qsz
    else:
        line_m = (v // _B) * hb + (v % hb)
        par_m = (v % _B) // hb
        if cov == vocab:
            line, par = line_m, par_m
        else:
            t = v - cov
            th = (vocab - cov) // SUB
            line_t = cov // SUB + (t % th)
            par_t = t // th
            tail = v >= cov
            line = jnp.where(tail, line_t, line_m)
            par = jnp.where(tail, par_t, par_m)
    return (line.reshape(BATCH // CHUNK, CHUNK),
            par.reshape(BATCH // CHUNK, CHUNK))


def kernel(user, item, context, W_user, W_item, W_context):
    wu, cov_u = _prep_table(W_user)
    wi, cov_i = _prep_table(W_item)
    wc, cov_c = _prep_table(W_context)
    u2, pu2 = _map_idx(user, 1000000, cov_u)
    i2, pi2 = _map_idx(item, 100000, cov_i)
    c2, pc2 = _map_idx(context, 1000, cov_c)
    out = _dict_embed(u2, pu2, i2, pi2, c2, pc2, wu, wi, wc)
    return out[:, :DIM]


# speedup vs baseline: 2.5784x; 1.2092x over previous
import jax
import jax.numpy as jnp
from jax import lax
from jax.experimental import pallas as pl
from jax.experimental.pallas import tpu as pltpu
from jax.experimental.pallas import tpu_sc as plsc

BATCH = 16384
DIM = 64
LINE = 256
SUB = LINE // DIM
LANES = 16
NUM_CORES = 2
NUM_SUBCORES = 16
NUM_WORKERS = NUM_CORES * NUM_SUBCORES
ROWS_PER_WORKER = BATCH // NUM_WORKERS
CHUNK = 64
NCHUNK = ROWS_PER_WORKER // CHUNK
GROUP = 16
_B = 8192


def _dict_embed_kernel(user_hbm, pu_hbm, item_hbm, pi_hbm, ctx_hbm, pc_hbm,
                       wu_hbm, wi_hbm, wc_hbm,
                       out_hbm, idx_u, idx_i, idx_c, par_u, par_i, par_c,
                       rows_u, rows_i, rows_c,
                       sem_u, sem_i, sem_c, sem_out):
    wid = lax.axis_index("s") * NUM_CORES + lax.axis_index("c")
    chunk_base = wid * NCHUNK
    row_base = wid * ROWS_PER_WORKER

    for (src, dst) in ((user_hbm, idx_u), (item_hbm, idx_i), (ctx_hbm, idx_c),
                       (pu_hbm, par_u), (pi_hbm, par_i), (pc_hbm, par_c)):
        pltpu.sync_copy(src.at[pl.ds(chunk_base, NCHUNK)], dst)

    def fire(j):
        sl = pl.ds((j % 2) * CHUNK, CHUNK)
        return (pltpu.async_copy(wu_hbm.at[idx_u.at[j]], rows_u.at[sl], sem_u),
                pltpu.async_copy(wi_hbm.at[idx_i.at[j]], rows_i.at[sl], sem_i),
                pltpu.async_copy(wc_hbm.at[idx_c.at[j]], rows_c.at[sl], sem_c))

    gathers = {0: fire(0)}
    stores = {}
    for j in range(NCHUNK):
        if j + 1 < NCHUNK:
            if j - 1 in stores:
                stores.pop(j - 1).wait()
            gathers[j + 1] = fire(j + 1)
        for c in gathers.pop(j):
            c.wait()
        base = (j % 2) * CHUNK

        def body(g, _):
            r0 = base + g * GROUP
            pv_u = par_u[j, pl.ds(g * GROUP, GROUP)]
            pv_i = par_i[j, pl.ds(g * GROUP, GROUP)]
            pv_c = par_c[j, pl.ds(g * GROUP, GROUP)]
            for k in range(GROUP):
                du = pv_u[k] * DIM
                di = pv_i[k] * DIM
                dc = pv_c[k] * DIM
                for c in range(DIM // LANES):
                    o = c * LANES
                    s = (rows_u[r0 + k, pl.ds(du + o, LANES)]
                         + rows_i[r0 + k, pl.ds(di + o, LANES)]
                         + rows_c[r0 + k, pl.ds(dc + o, LANES)])
                    rows_u[r0 + k, pl.ds(o, LANES)] = s
            return _

        lax.fori_loop(0, CHUNK // GROUP, body, 0)

        stores[j] = pltpu.async_copy(
            rows_u.at[pl.ds(base, CHUNK)],
            out_hbm.at[pl.ds(row_base + j * CHUNK, CHUNK)],
            sem_out)
    for s in stores.values():
        s.wait()


def _dict_embed(u2, pu2, i2, pi2, c2, pc2, wu, wi, wc):
    mesh = plsc.VectorSubcoreMesh(core_axis_name="c", subcore_axis_name="s")
    return pl.kernel(
        _dict_embed_kernel,
        mesh=mesh,
        out_type=jax.ShapeDtypeStruct((BATCH, LINE), jnp.float32),
        scratch_types=[
            pltpu.VMEM((NCHUNK, CHUNK), jnp.int32),
            pltpu.VMEM((NCHUNK, CHUNK), jnp.int32),
            pltpu.VMEM((NCHUNK, CHUNK), jnp.int32),
            pltpu.VMEM((NCHUNK, CHUNK), jnp.int32),
            pltpu.VMEM((NCHUNK, CHUNK), jnp.int32),
            pltpu.VMEM((NCHUNK, CHUNK), jnp.int32),
            pltpu.VMEM((2 * CHUNK, LINE), jnp.float32),
            pltpu.VMEM((2 * CHUNK, LINE), jnp.float32),
            pltpu.VMEM((2 * CHUNK, LINE), jnp.float32),
            pltpu.SemaphoreType.DMA,
            pltpu.SemaphoreType.DMA,
            pltpu.SemaphoreType.DMA,
            pltpu.SemaphoreType.DMA,
        ],
    )(u2, pu2, i2, pi2, c2, pc2, wu, wi, wc)


def _pack_body(x_ref, out_ref):
    x = x_ref[...]
    q = x.shape[1] // SUB
    x4 = jnp.concatenate([x[:, i * q:(i + 1) * q] for i in range(SUB)],
                         axis=0)
    hi = x4.astype(jnp.bfloat16)
    lo = (x4 - hi.astype(jnp.float32)).astype(jnp.bfloat16)
    eye = jnp.eye(LINE, dtype=jnp.bfloat16)
    dims = (((0,), (0,)), ((), ()))
    out_ref[...] = (
        lax.dot_general(hi, eye, dims, preferred_element_type=jnp.float32)
        + lax.dot_general(lo, eye, dims, preferred_element_type=jnp.float32))


def _pack_main(wt, nblk, block):
    return pl.pallas_call(
        _pack_body,
        grid=(nblk,),
        in_specs=[pl.BlockSpec((DIM, block), lambda j: (0, j))],
        out_specs=pl.BlockSpec((block // SUB, LINE), lambda j: (j, 0)),
        out_shape=jax.ShapeDtypeStruct((wt.shape[1] // SUB, LINE),
                                       jnp.float32),
    )(wt)


def _pack_whole(wt):
    return pl.pallas_call(
        _pack_body,
        out_shape=jax.ShapeDtypeStruct((wt.shape[1] // SUB, LINE),
                                       jnp.float32),
    )(wt)


def _prep_table(w):
    v = w.shape[0]
    nblk = v // _B
    cov = nblk * _B
    wt = w.T
    if v <= _B:
        packed, cov = _pack_whole(wt), v
    elif cov == v:
        packed = _pack_main(wt, nblk, _B)
    else:
        main = _pack_main(wt, nblk, _B)
        tail = _pack_whole(w[cov:].T)
        packed = jax.lax.dynamic_update_slice(main, tail, (cov // SUB, 0))
    return packed, cov


def _map_idx(idx, vocab, cov):
    v = idx.astype(jnp.int32)
    hb = _B // SUB
    if cov == vocab and vocab <= _B:
        qsz = vocab // SUB
        line = v % qsz
        par = v // qsz
    else:
        line_m = (v // _B) * hb + (v % hb)
        par_m = (v % _B) // hb
        if cov == vocab:
            line, par = line_m, par_m
        else:
            t = v - cov
            th = (vocab - cov) // SUB
            line_t = cov // SUB + (t % th)
            par_t = t // th
            tail = v >= cov
            line = jnp.where(tail, line_t, line_m)
            par = jnp.where(tail, par_t, par_m)
    return (line.reshape(BATCH // CHUNK, CHUNK),
            par.reshape(BATCH // CHUNK, CHUNK))


def kernel(user, item, context, W_user, W_item, W_context):
    wu, cov_u = _prep_table(W_user)
    wi, cov_i = _prep_table(W_item)
    wc, cov_c = _prep_table(W_context)
    u2, pu2 = _map_idx(user, 1000000, cov_u)
    i2, pi2 = _map_idx(item, 100000, cov_i)
    c2, pc2 = _map_idx(context, 1000, cov_c)
    out = _dict_embed(u2, pu2, i2, pi2, c2, pc2, wu, wi, wc)
    return out[:, :DIM]


# speedup vs baseline: 2.9727x; 1.1529x over previous
import jax
import jax.numpy as jnp
from jax import lax
from jax.experimental import pallas as pl
from jax.experimental.pallas import tpu as pltpu
from jax.experimental.pallas import tpu_sc as plsc

BATCH = 16384
DIM = 64
LINE = 256
SUB = LINE // DIM
LANES = 16
NUM_CORES = 2
NUM_SUBCORES = 16
NUM_WORKERS = NUM_CORES * NUM_SUBCORES
ROWS_PER_WORKER = BATCH // NUM_WORKERS
CHUNK = 64
NCHUNK = ROWS_PER_WORKER // CHUNK
GROUP = 16
_B = 16384


def _dict_embed_kernel(user_hbm, pu_hbm, item_hbm, pi_hbm, ctx_hbm, pc_hbm,
                       wu_hbm, wi_hbm, wc_hbm,
                       out_hbm, idx_u, idx_i, idx_c, par_u, par_i, par_c,
                       rows_u, rows_i, rows_c,
                       sem_u, sem_i, sem_c, sem_out):
    wid = lax.axis_index("s") * NUM_CORES + lax.axis_index("c")
    chunk_base = wid * NCHUNK
    row_base = wid * ROWS_PER_WORKER

    for (src, dst) in ((user_hbm, idx_u), (item_hbm, idx_i), (ctx_hbm, idx_c),
                       (pu_hbm, par_u), (pi_hbm, par_i), (pc_hbm, par_c)):
        pltpu.sync_copy(src.at[pl.ds(chunk_base, NCHUNK)], dst)

    def fire(j):
        sl = pl.ds((j % 2) * CHUNK, CHUNK)
        return (pltpu.async_copy(wu_hbm.at[idx_u.at[j]], rows_u.at[sl], sem_u),
                pltpu.async_copy(wi_hbm.at[idx_i.at[j]], rows_i.at[sl], sem_i),
                pltpu.async_copy(wc_hbm.at[idx_c.at[j]], rows_c.at[sl], sem_c))

    gathers = {0: fire(0)}
    stores = {}
    for j in range(NCHUNK):
        if j + 1 < NCHUNK:
            if j - 1 in stores:
                stores.pop(j - 1).wait()
            gathers[j + 1] = fire(j + 1)
        for c in gathers.pop(j):
            c.wait()
        base = (j % 2) * CHUNK

        def body(g, _):
            r0 = base + g * GROUP
            pv_u = par_u[j, pl.ds(g * GROUP, GROUP)]
            pv_i = par_i[j, pl.ds(g * GROUP, GROUP)]
            pv_c = par_c[j, pl.ds(g * GROUP, GROUP)]
            for k in range(GROUP):
                du = pv_u[k] * DIM
                di = pv_i[k] * DIM
                dc = pv_c[k] * DIM
                for c in range(DIM // LANES):
                    o = c * LANES
                    s = (rows_u[r0 + k, pl.ds(du + o, LANES)]
                         + rows_i[r0 + k, pl.ds(di + o, LANES)]
                         + rows_c[r0 + k, pl.ds(dc + o, LANES)])
                    rows_u[r0 + k, pl.ds(o, LANES)] = s
            return _

        lax.fori_loop(0, CHUNK // GROUP, body, 0)

        stores[j] = pltpu.async_copy(
            rows_u.at[pl.ds(base, CHUNK)],
            out_hbm.at[pl.ds(row_base + j * CHUNK, CHUNK)],
            sem_out)
    for s in stores.values():
        s.wait()


def _dict_embed(u2, pu2, i2, pi2, c2, pc2, wu, wi, wc):
    mesh = plsc.VectorSubcoreMesh(core_axis_name="c", subcore_axis_name="s")
    return pl.kernel(
        _dict_embed_kernel,
        mesh=mesh,
        out_type=jax.ShapeDtypeStruct((BATCH, LINE), jnp.float32),
        scratch_types=[
            pltpu.VMEM((NCHUNK, CHUNK), jnp.int32),
            pltpu.VMEM((NCHUNK, CHUNK), jnp.int32),
            pltpu.VMEM((NCHUNK, CHUNK), jnp.int32),
            pltpu.VMEM((NCHUNK, CHUNK), jnp.int32),
            pltpu.VMEM((NCHUNK, CHUNK), jnp.int32),
            pltpu.VMEM((NCHUNK, CHUNK), jnp.int32),
            pltpu.VMEM((2 * CHUNK, LINE), jnp.float32),
            pltpu.VMEM((2 * CHUNK, LINE), jnp.float32),
            pltpu.VMEM((2 * CHUNK, LINE), jnp.float32),
            pltpu.SemaphoreType.DMA,
            pltpu.SemaphoreType.DMA,
            pltpu.SemaphoreType.DMA,
            pltpu.SemaphoreType.DMA,
        ],
    )(u2, pu2, i2, pi2, c2, pc2, wu, wi, wc)


def _pack_body(x_ref, out_ref):
    x = x_ref[...]
    q = x.shape[1] // SUB
    x4 = jnp.concatenate([x[:, i * q:(i + 1) * q] for i in range(SUB)],
                         axis=0)
    hi = x4.astype(jnp.bfloat16)
    lo = (x4 - hi.astype(jnp.float32)).astype(jnp.bfloat16)
    eye = jnp.eye(LINE, dtype=jnp.bfloat16)
    dims = (((0,), (0,)), ((), ()))
    out_ref[...] = (
        lax.dot_general(hi, eye, dims, preferred_element_type=jnp.float32)
        + lax.dot_general(lo, eye, dims, preferred_element_type=jnp.float32))


def _pack_main(wt, nblk, block):
    return pl.pallas_call(
        _pack_body,
        grid=(nblk,),
        in_specs=[pl.BlockSpec((DIM, block), lambda j: (0, j))],
        out_specs=pl.BlockSpec((block // SUB, LINE), lambda j: (j, 0)),
        out_shape=jax.ShapeDtypeStruct((wt.shape[1] // SUB, LINE),
                                       jnp.float32),
    )(wt)


def _pack_whole(wt):
    return pl.pallas_call(
        _pack_body,
        out_shape=jax.ShapeDtypeStruct((wt.shape[1] // SUB, LINE),
                                       jnp.float32),
    )(wt)


def _prep_table(w):
    v = w.shape[0]
    nblk = v // _B
    cov = nblk * _B
    wt = w.T
    if v <= _B:
        packed, cov = _pack_whole(wt), v
    elif cov == v:
        packed = _pack_main(wt, nblk, _B)
    else:
        main = _pack_main(wt, nblk, _B)
        tail = _pack_whole(w[cov:].T)
        packed = jax.lax.dynamic_update_slice(main, tail, (cov // SUB, 0))
    return packed, cov


def _map_idx(idx, vocab, cov):
    v = idx.astype(jnp.int32)
    hb = _B // SUB
    if cov == vocab and vocab <= _B:
        qsz = vocab // SUB
        line = v % qsz
        par = v // qsz
    else:
        line_m = (v // _B) * hb + (v % hb)
        par_m = (v % _B) // hb
        if cov == vocab:
            line, par = line_m, par_m
        else:
            t = v - cov
            th = (vocab - cov) // SUB
            line_t = cov // SUB + (t % th)
            par_t = t // th
            tail = v >= cov
            line = jnp.where(tail, line_t, line_m)
            par = jnp.where(tail, par_t, par_m)
    return (line.reshape(BATCH // CHUNK, CHUNK),
            par.reshape(BATCH // CHUNK, CHUNK))


def kernel(user, item, context, W_user, W_item, W_context):
    wu, cov_u = _prep_table(W_user)
    wi, cov_i = _prep_table(W_item)
    wc, cov_c = _prep_table(W_context)
    u2, pu2 = _map_idx(user, 1000000, cov_u)
    i2, pi2 = _map_idx(item, 100000, cov_i)
    c2, pc2 = _map_idx(context, 1000, cov_c)
    out = _dict_embed(u2, pu2, i2, pi2, c2, pc2, wu, wi, wc)
    return out[:, :DIM]
